# Initial kernel scaffold; baseline (speedup 1.0000x reference)
import jax
import jax.numpy as jnp
from jax.experimental import pallas as pl


def kernel(embeddings, visibility_scores, bbox_ltwh, keypoints_xyc, feats_masks, W, b):
    raise NotImplementedError("write your pallas kernel here")


# speedup vs baseline: 2.9813x; 2.9813x over previous
import jax
import jax.numpy as jnp
from jax.experimental import pallas as pl

_EMB = 3072
_SMALL = 61
_SMALL_PAD = 64
_N = 1024
_M_BLK = 512


def _proj_kernel(x_ref, s_ref, we_ref, ws_ref, b_ref, m_ref, o_ref):
    x = x_ref[...].astype(jnp.bfloat16)
    acc = jax.lax.dot_general(
        x, we_ref[...], (((1,), (0,)), ((), ())),
        preferred_element_type=jnp.float32)
    acc += jax.lax.dot_general(
        s_ref[...].astype(jnp.bfloat16), ws_ref[...], (((1,), (0,)), ((), ())),
        preferred_element_type=jnp.float32)
    o_ref[...] = (acc + b_ref[...]) * m_ref[...]


def kernel(embeddings, visibility_scores, bbox_ltwh, keypoints_xyc, feats_masks, W, b):
    bsz, slen = feats_masks.shape
    m_rows = bsz * slen

    x = embeddings.reshape(m_rows, _EMB)
    small = jnp.concatenate(
        [visibility_scores.reshape(m_rows, 6),
         bbox_ltwh.reshape(m_rows, 4),
         keypoints_xyc.reshape(m_rows, 51),
         jnp.zeros((m_rows, _SMALL_PAD - _SMALL), jnp.float32)],
        axis=-1)
    mask = feats_masks.reshape(m_rows, 1).astype(jnp.float32)

    wt = W.T
    w_emb = wt[:_EMB].astype(jnp.bfloat16)
    w_small = jnp.concatenate(
        [wt[_EMB:], jnp.zeros((_SMALL_PAD - _SMALL, _N), jnp.float32)],
        axis=0).astype(jnp.bfloat16)
    bias = b.reshape(1, _N)

    grid = (m_rows // _M_BLK,)
    out = pl.pallas_call(
        _proj_kernel,
        grid=grid,
        in_specs=[
            pl.BlockSpec((_M_BLK, _EMB), lambda m: (m, 0)),
            pl.BlockSpec((_M_BLK, _SMALL_PAD), lambda m: (m, 0)),
            pl.BlockSpec((_EMB, _N), lambda m: (0, 0)),
            pl.BlockSpec((_SMALL_PAD, _N), lambda m: (0, 0)),
            pl.BlockSpec((1, _N), lambda m: (0, 0)),
            pl.BlockSpec((_M_BLK, 1), lambda m: (m, 0)),
        ],
        out_specs=pl.BlockSpec((_M_BLK, _N), lambda m: (m, 0)),
        out_shape=jax.ShapeDtypeStruct((m_rows, _N), jnp.float32),
    )(x, small, w_emb, w_small, bias, mask)

    return out.reshape(bsz, slen, _N)
